# Initial kernel scaffold; baseline (speedup 1.0000x reference)
import functools

import jax
import jax.numpy as jnp
from jax import lax
from jax.experimental import pallas as pl
from jax.experimental.pallas import tpu as pltpu
from jax.experimental.pallas import tpu_sc as plsc

NC = 2
NS = 16
LANES = 16

NNODES = 10000
NPAD = 10240
HID = 64
CHUNK = 128
CH_PER_TILE = 80
EPAD = NC * NS * CH_PER_TILE * CHUNK
ROWS_PER_TILE = NPAD // NS
BR = 512
DEGW = 16


def _sc_mesh():
    return plsc.VectorSubcoreMesh(
        core_axis_name="c", subcore_axis_name="s", num_cores=NC, num_subcores=NS
    )


def _sc_degree(dst_r):
    @functools.partial(
        pl.kernel,
        out_type=jax.ShapeDtypeStruct((NC, NPAD, DEGW), jnp.float32),
        mesh=_sc_mesh(),
        scratch_types=[
            pltpu.VMEM((CH_PER_TILE, CHUNK), jnp.int32),
            pltpu.VMEM((CHUNK, DEGW), jnp.float32),
            pltpu.VMEM((ROWS_PER_TILE, DEGW), jnp.float32),
            pltpu.VMEM_SHARED((NPAD, DEGW), jnp.float32),
        ],
    )
    def deg_kernel(dst_hbm, out_hbm, dst_v, ones_v, zero_v, acc_sh):
        c = lax.axis_index("c")
        s = lax.axis_index("s")

        def fill_ones(i, carry):
            ones_v[i, :] = jnp.ones((LANES,), jnp.float32)
            return carry

        lax.fori_loop(0, CHUNK, fill_ones, 0)

        def fill_zero(i, carry):
            zero_v[i, :] = jnp.zeros((LANES,), jnp.float32)
            return carry

        lax.fori_loop(0, ROWS_PER_TILE, fill_zero, 0)
        pltpu.sync_copy(zero_v, acc_sh.at[pl.ds(s * ROWS_PER_TILE, ROWS_PER_TILE)])
        plsc.subcore_barrier()

        pltpu.sync_copy(dst_hbm.at[c, s], dst_v)

        def body(j, carry):
            pltpu.sync_copy(ones_v, acc_sh.at[dst_v.at[j]], add=True)
            return carry

        lax.fori_loop(0, CH_PER_TILE, body, 0)
        plsc.subcore_barrier()
        pltpu.sync_copy(
            acc_sh.at[pl.ds(s * ROWS_PER_TILE, ROWS_PER_TILE)],
            out_hbm.at[c, pl.ds(s * ROWS_PER_TILE, ROWS_PER_TILE)],
        )

    return deg_kernel(dst_r)


def _sc_aggregate(g, src_r, dst_r):
    @functools.partial(
        pl.kernel,
        out_type=jax.ShapeDtypeStruct((NC, NPAD, HID), jnp.float32),
        mesh=_sc_mesh(),
        scratch_types=[
            pltpu.VMEM((CH_PER_TILE, CHUNK), jnp.int32),
            pltpu.VMEM((CH_PER_TILE, CHUNK), jnp.int32),
            pltpu.VMEM((CHUNK, HID), jnp.float32),
            pltpu.VMEM((ROWS_PER_TILE, HID), jnp.float32),
            pltpu.VMEM_SHARED((NPAD, HID), jnp.float32),
            pltpu.SemaphoreType.DMA,
        ],
    )
    def agg_kernel(g_hbm, src_hbm, dst_hbm, out_hbm, src_v, dst_v, rows_v, zero_v,
                   acc_sh, sem):
        c = lax.axis_index("c")
        s = lax.axis_index("s")

        def fill_zero(i, carry):
            for k in range(HID // LANES):
                zero_v[i, pl.ds(k * LANES, LANES)] = jnp.zeros((LANES,), jnp.float32)
            return carry

        lax.fori_loop(0, ROWS_PER_TILE, fill_zero, 0)
        pltpu.sync_copy(zero_v, acc_sh.at[pl.ds(s * ROWS_PER_TILE, ROWS_PER_TILE)])
        plsc.subcore_barrier()

        pltpu.sync_copy(src_hbm.at[c, s], src_v)
        pltpu.sync_copy(dst_hbm.at[c, s], dst_v)

        def body(j, carry):
            pltpu.async_copy(g_hbm.at[src_v.at[j]], rows_v, sem).wait()
            pltpu.sync_copy(rows_v, acc_sh.at[dst_v.at[j]], add=True)
            return carry

        lax.fori_loop(0, CH_PER_TILE, body, 0)
        plsc.subcore_barrier()
        pltpu.sync_copy(
            acc_sh.at[pl.ds(s * ROWS_PER_TILE, ROWS_PER_TILE)],
            out_hbm.at[c, pl.ds(s * ROWS_PER_TILE, ROWS_PER_TILE)],
        )

    return agg_kernel(g, src_r, dst_r)


def _dinv_from(deg_ref):
    deg = deg_ref[0, :, 0] + deg_ref[1, :, 0]
    return lax.rsqrt(deg + 1.0)


def _first_body(x_ref, deg_ref, w1_ref, out_ref):
    dinv = _dinv_from(deg_ref)
    h = jnp.dot(x_ref[...], w1_ref[...], preferred_element_type=jnp.float32)
    out_ref[...] = h * dinv[:, None]


def _tc_first(x_pad, deg2, W1):
    return pl.pallas_call(
        _first_body,
        grid=(NPAD // BR,),
        in_specs=[
            pl.BlockSpec((BR, 128), lambda i: (i, 0)),
            pl.BlockSpec((NC, BR, DEGW), lambda i: (0, i, 0)),
            pl.BlockSpec((128, HID), lambda i: (0, 0)),
        ],
        out_specs=pl.BlockSpec((BR, HID), lambda i: (i, 0)),
        out_shape=jax.ShapeDtypeStruct((NPAD, HID), jnp.float32),
    )(x_pad, deg2, W1)


def _mid_body(p_ref, g1_ref, deg_ref, w2_ref, b1_ref, out_ref):
    dinv = _dinv_from(deg_ref)
    pp = p_ref[0] + p_ref[1] + g1_ref[...]
    t = jnp.maximum(pp * dinv[:, None] + b1_ref[...], 0.0)
    g2 = jnp.dot(t, w2_ref[...], preferred_element_type=jnp.float32) * dinv[:, None]
    row = pl.program_id(0) * BR + lax.broadcasted_iota(jnp.int32, (BR, 1), 0)
    out_ref[...] = jnp.where(row < NNODES, g2, 0.0)


def _tc_mid(p, g1, deg2, W2, b1r):
    return pl.pallas_call(
        _mid_body,
        grid=(NPAD // BR,),
        in_specs=[
            pl.BlockSpec((NC, BR, HID), lambda i: (0, i, 0)),
            pl.BlockSpec((BR, HID), lambda i: (i, 0)),
            pl.BlockSpec((NC, BR, DEGW), lambda i: (0, i, 0)),
            pl.BlockSpec((HID, HID), lambda i: (0, 0)),
            pl.BlockSpec((1, HID), lambda i: (0, 0)),
        ],
        out_specs=pl.BlockSpec((BR, HID), lambda i: (i, 0)),
        out_shape=jax.ShapeDtypeStruct((NPAD, HID), jnp.float32),
    )(p, g1, deg2, W2, b1r)


def _final_body(q_ref, g2_ref, deg_ref, b2_ref, w3_ref, b3_ref, out_ref):
    dinv = _dinv_from(deg_ref)
    qq = q_ref[0] + q_ref[1] + g2_ref[...]
    t = jnp.maximum(qq * dinv[:, None] + b2_ref[...], 0.0)
    o = jnp.sum(t * w3_ref[...], axis=1, keepdims=True) + b3_ref[0, 0]
    out_ref[...] = o


def _tc_final(q, g2, deg2, b2r, w3r, b3r):
    return pl.pallas_call(
        _final_body,
        grid=(NPAD // BR,),
        in_specs=[
            pl.BlockSpec((NC, BR, HID), lambda i: (0, i, 0)),
            pl.BlockSpec((BR, HID), lambda i: (i, 0)),
            pl.BlockSpec((NC, BR, DEGW), lambda i: (0, i, 0)),
            pl.BlockSpec((1, HID), lambda i: (0, 0)),
            pl.BlockSpec((1, HID), lambda i: (0, 0)),
            pl.BlockSpec((1, HID), lambda i: (0, 0)),
        ],
        out_specs=pl.BlockSpec((BR, 1), lambda i: (i, 0)),
        out_shape=jax.ShapeDtypeStruct((NPAD, 1), jnp.float32),
    )(q, g2, deg2, b2r, w3r, b3r)


def kernel(x, edge_index, W1, b1, W2, b2, W3, b3):
    n, _ = x.shape
    e = edge_index.shape[1]
    x_pad = jnp.pad(x, ((0, NPAD - n), (0, 0)))
    ep = jnp.pad(edge_index, ((0, 0), (0, EPAD - e)), constant_values=n)
    src_r = ep[0].reshape(NC, NS, CH_PER_TILE, CHUNK)
    dst_r = ep[1].reshape(NC, NS, CH_PER_TILE, CHUNK)

    deg2 = _sc_degree(dst_r)
    g1 = _tc_first(x_pad, deg2, W1)
    p = _sc_aggregate(g1, src_r, dst_r)
    g2 = _tc_mid(p, g1, deg2, W2, b1.reshape(1, HID))
    q = _sc_aggregate(g2, src_r, dst_r)
    res = _tc_final(
        q, g2, deg2,
        b2.reshape(1, HID),
        W3.reshape(1, HID),
        jnp.broadcast_to(b3.reshape(1, 1), (1, HID)),
    )
    return res[:n, 0]


# speedup vs baseline: 15.1488x; 15.1488x over previous
import functools

import jax
import jax.numpy as jnp
from jax import lax
from jax.experimental import pallas as pl
from jax.experimental.pallas import tpu as pltpu
from jax.experimental.pallas import tpu_sc as plsc

NC = 2
NS = 16
LANES = 16

NNODES = 10000
NPAD = 10240
HID = 64
CHUNK = 128
CH_PER_TILE = 80
EPAD = NC * NS * CH_PER_TILE * CHUNK
ROWS_PER_TILE = NPAD // NS
BR = 512
DEGW = 16


def _sc_mesh():
    return plsc.VectorSubcoreMesh(
        core_axis_name="c", subcore_axis_name="s", num_cores=NC, num_subcores=NS
    )


_SC_PARAMS = pltpu.CompilerParams(use_tc_tiling_on_sc=False)


def _sc_degree(dst_r):
    @functools.partial(
        pl.kernel,
        out_type=jax.ShapeDtypeStruct((NC, NPAD, DEGW), jnp.float32),
        mesh=_sc_mesh(),
        scratch_types=[
            pltpu.VMEM((CH_PER_TILE, CHUNK), jnp.int32),
            pltpu.VMEM((CHUNK, DEGW), jnp.float32),
            pltpu.VMEM((ROWS_PER_TILE, DEGW), jnp.float32),
            pltpu.VMEM_SHARED((NPAD, DEGW), jnp.float32),
        ],
        compiler_params=_SC_PARAMS,
    )
    def deg_kernel(dst_hbm, out_hbm, dst_v, ones_v, zero_v, acc_sh):
        c = lax.axis_index("c")
        s = lax.axis_index("s")

        def fill_ones(i, carry):
            ones_v[i, :] = jnp.ones((LANES,), jnp.float32)
            return carry

        lax.fori_loop(0, CHUNK, fill_ones, 0)

        def fill_zero(i, carry):
            zero_v[i, :] = jnp.zeros((LANES,), jnp.float32)
            return carry

        lax.fori_loop(0, ROWS_PER_TILE, fill_zero, 0)
        pltpu.sync_copy(zero_v, acc_sh.at[pl.ds(s * ROWS_PER_TILE, ROWS_PER_TILE)])
        plsc.subcore_barrier()

        pltpu.sync_copy(dst_hbm.at[c, s], dst_v)

        def body(j, carry):
            pltpu.sync_copy(ones_v, acc_sh.at[dst_v.at[j]], add=True)
            return carry

        lax.fori_loop(0, CH_PER_TILE, body, 0)
        plsc.subcore_barrier()
        pltpu.sync_copy(
            acc_sh.at[pl.ds(s * ROWS_PER_TILE, ROWS_PER_TILE)],
            out_hbm.at[c, pl.ds(s * ROWS_PER_TILE, ROWS_PER_TILE)],
        )

    return deg_kernel(dst_r)


def _sc_aggregate(g, src_r, dst_r):
    @functools.partial(
        pl.kernel,
        out_type=jax.ShapeDtypeStruct((NC, NPAD, HID), jnp.float32),
        mesh=_sc_mesh(),
        scratch_types=[
            pltpu.VMEM((CH_PER_TILE, CHUNK), jnp.int32),
            pltpu.VMEM((CH_PER_TILE, CHUNK), jnp.int32),
            pltpu.VMEM((CHUNK, HID), jnp.float32),
            pltpu.VMEM((ROWS_PER_TILE, HID), jnp.float32),
            pltpu.VMEM_SHARED((NPAD, HID), jnp.float32),
            pltpu.SemaphoreType.DMA,
        ],
        compiler_params=_SC_PARAMS,
    )
    def agg_kernel(g_hbm, src_hbm, dst_hbm, out_hbm, src_v, dst_v, rows_v, zero_v,
                   acc_sh, sem):
        c = lax.axis_index("c")
        s = lax.axis_index("s")

        def fill_zero(i, carry):
            for k in range(HID // LANES):
                zero_v[i, pl.ds(k * LANES, LANES)] = jnp.zeros((LANES,), jnp.float32)
            return carry

        lax.fori_loop(0, ROWS_PER_TILE, fill_zero, 0)
        pltpu.sync_copy(zero_v, acc_sh.at[pl.ds(s * ROWS_PER_TILE, ROWS_PER_TILE)])
        plsc.subcore_barrier()

        pltpu.sync_copy(src_hbm.at[c, s], src_v)
        pltpu.sync_copy(dst_hbm.at[c, s], dst_v)

        def body(j, carry):
            pltpu.async_copy(g_hbm.at[src_v.at[j]], rows_v, sem).wait()
            pltpu.sync_copy(rows_v, acc_sh.at[dst_v.at[j]], add=True)
            return carry

        lax.fori_loop(0, CH_PER_TILE, body, 0)
        plsc.subcore_barrier()
        pltpu.sync_copy(
            acc_sh.at[pl.ds(s * ROWS_PER_TILE, ROWS_PER_TILE)],
            out_hbm.at[c, pl.ds(s * ROWS_PER_TILE, ROWS_PER_TILE)],
        )

    return agg_kernel(g, src_r, dst_r)


def _dinv_from(deg_ref):
    deg = deg_ref[0, :, 0] + deg_ref[1, :, 0]
    return lax.rsqrt(deg + 1.0)


def _first_body(x_ref, deg_ref, w1_ref, out_ref):
    dinv = _dinv_from(deg_ref)
    h = jnp.dot(x_ref[...], w1_ref[...], preferred_element_type=jnp.float32)
    out_ref[...] = h * dinv[:, None]


def _tc_first(x_pad, deg2, W1):
    return pl.pallas_call(
        _first_body,
        grid=(NPAD // BR,),
        in_specs=[
            pl.BlockSpec((BR, 128), lambda i: (i, 0)),
            pl.BlockSpec((NC, BR, DEGW), lambda i: (0, i, 0)),
            pl.BlockSpec((128, HID), lambda i: (0, 0)),
        ],
        out_specs=pl.BlockSpec((BR, HID), lambda i: (i, 0)),
        out_shape=jax.ShapeDtypeStruct((NPAD, HID), jnp.float32),
    )(x_pad, deg2, W1)


def _mid_body(p_ref, g1_ref, deg_ref, w2_ref, b1_ref, out_ref):
    dinv = _dinv_from(deg_ref)
    pp = p_ref[0] + p_ref[1] + g1_ref[...]
    t = jnp.maximum(pp * dinv[:, None] + b1_ref[...], 0.0)
    g2 = jnp.dot(t, w2_ref[...], preferred_element_type=jnp.float32) * dinv[:, None]
    row = pl.program_id(0) * BR + lax.broadcasted_iota(jnp.int32, (BR, 1), 0)
    out_ref[...] = jnp.where(row < NNODES, g2, 0.0)


def _tc_mid(p, g1, deg2, W2, b1r):
    return pl.pallas_call(
        _mid_body,
        grid=(NPAD // BR,),
        in_specs=[
            pl.BlockSpec((NC, BR, HID), lambda i: (0, i, 0)),
            pl.BlockSpec((BR, HID), lambda i: (i, 0)),
            pl.BlockSpec((NC, BR, DEGW), lambda i: (0, i, 0)),
            pl.BlockSpec((HID, HID), lambda i: (0, 0)),
            pl.BlockSpec((1, HID), lambda i: (0, 0)),
        ],
        out_specs=pl.BlockSpec((BR, HID), lambda i: (i, 0)),
        out_shape=jax.ShapeDtypeStruct((NPAD, HID), jnp.float32),
    )(p, g1, deg2, W2, b1r)


def _final_body(q_ref, g2_ref, deg_ref, b2_ref, w3_ref, b3_ref, out_ref):
    dinv = _dinv_from(deg_ref)
    qq = q_ref[0] + q_ref[1] + g2_ref[...]
    t = jnp.maximum(qq * dinv[:, None] + b2_ref[...], 0.0)
    o = jnp.sum(t * w3_ref[...], axis=1, keepdims=True) + b3_ref[0, 0]
    out_ref[...] = o


def _tc_final(q, g2, deg2, b2r, w3r, b3r):
    return pl.pallas_call(
        _final_body,
        grid=(NPAD // BR,),
        in_specs=[
            pl.BlockSpec((NC, BR, HID), lambda i: (0, i, 0)),
            pl.BlockSpec((BR, HID), lambda i: (i, 0)),
            pl.BlockSpec((NC, BR, DEGW), lambda i: (0, i, 0)),
            pl.BlockSpec((1, HID), lambda i: (0, 0)),
            pl.BlockSpec((1, HID), lambda i: (0, 0)),
            pl.BlockSpec((1, HID), lambda i: (0, 0)),
        ],
        out_specs=pl.BlockSpec((BR, 1), lambda i: (i, 0)),
        out_shape=jax.ShapeDtypeStruct((NPAD, 1), jnp.float32),
    )(q, g2, deg2, b2r, w3r, b3r)


def kernel(x, edge_index, W1, b1, W2, b2, W3, b3):
    n, _ = x.shape
    e = edge_index.shape[1]
    x_pad = jnp.pad(x, ((0, NPAD - n), (0, 0)))
    ep = jnp.pad(edge_index, ((0, 0), (0, EPAD - e)), constant_values=n)
    src_r = ep[0].reshape(NC, NS, CH_PER_TILE, CHUNK)
    dst_r = ep[1].reshape(NC, NS, CH_PER_TILE, CHUNK)

    deg2 = _sc_degree(dst_r)
    g1 = _tc_first(x_pad, deg2, W1)
    p = _sc_aggregate(g1, src_r, dst_r)
    g2 = _tc_mid(p, g1, deg2, W2, b1.reshape(1, HID))
    q = _sc_aggregate(g2, src_r, dst_r)
    res = _tc_final(
        q, g2, deg2,
        b2.reshape(1, HID),
        W3.reshape(1, HID),
        jnp.broadcast_to(b3.reshape(1, 1), (1, HID)),
    )
    return res[:n, 0]


# speedup vs baseline: 17.5362x; 1.1576x over previous
import functools

import jax
import jax.numpy as jnp
from jax import lax
from jax.experimental import pallas as pl
from jax.experimental.pallas import tpu as pltpu
from jax.experimental.pallas import tpu_sc as plsc

NC = 2
NS = 16
LANES = 16

NNODES = 10000
NPAD = 10240
HID = 64
CHUNK = 128
CH_PER_TILE = 80
EPAD = NC * NS * CH_PER_TILE * CHUNK
ROWS_PER_TILE = NPAD // NS
BR = 512
DEGW = 16
NBUF = 4


def _sc_mesh():
    return plsc.VectorSubcoreMesh(
        core_axis_name="c", subcore_axis_name="s", num_cores=NC, num_subcores=NS
    )


_SC_PARAMS = pltpu.CompilerParams(use_tc_tiling_on_sc=False)


def _sc_degree(dst_r):
    @functools.partial(
        pl.kernel,
        out_type=jax.ShapeDtypeStruct((NC, NPAD, DEGW), jnp.float32),
        mesh=_sc_mesh(),
        scratch_types=[
            pltpu.VMEM((CH_PER_TILE, CHUNK), jnp.int32),
            pltpu.VMEM((CHUNK, DEGW), jnp.float32),
            pltpu.VMEM((ROWS_PER_TILE, DEGW), jnp.float32),
            pltpu.VMEM_SHARED((NPAD, DEGW), jnp.float32),
            pltpu.SemaphoreType.DMA,
        ],
        compiler_params=_SC_PARAMS,
    )
    def deg_kernel(dst_hbm, out_hbm, dst_v, ones_v, zero_v, acc_sh, sem):
        c = lax.axis_index("c")
        s = lax.axis_index("s")

        def fill_ones(i, carry):
            ones_v[i, :] = jnp.ones((LANES,), jnp.float32)
            return carry

        lax.fori_loop(0, CHUNK, fill_ones, 0)

        def fill_zero(i, carry):
            zero_v[i, :] = jnp.zeros((LANES,), jnp.float32)
            return carry

        lax.fori_loop(0, ROWS_PER_TILE, fill_zero, 0)
        pltpu.sync_copy(zero_v, acc_sh.at[pl.ds(s * ROWS_PER_TILE, ROWS_PER_TILE)])
        plsc.subcore_barrier()

        pltpu.sync_copy(dst_hbm.at[c, s], dst_v)

        def body(j, carry):
            for k in range(8):
                pltpu.async_copy(ones_v, acc_sh.at[dst_v.at[j * 8 + k]], sem,
                                 add=True)
            for k in range(8):
                pltpu.make_async_copy(ones_v, acc_sh.at[dst_v.at[j * 8 + k]],
                                      sem).wait()
            return carry

        lax.fori_loop(0, CH_PER_TILE // 8, body, 0)
        plsc.subcore_barrier()
        pltpu.sync_copy(
            acc_sh.at[pl.ds(s * ROWS_PER_TILE, ROWS_PER_TILE)],
            out_hbm.at[c, pl.ds(s * ROWS_PER_TILE, ROWS_PER_TILE)],
        )

    return deg_kernel(dst_r)


def _sc_aggregate(g, src_r, dst_r):
    @functools.partial(
        pl.kernel,
        out_type=jax.ShapeDtypeStruct((NC, NPAD, HID), jnp.float32),
        mesh=_sc_mesh(),
        scratch_types=[
            pltpu.VMEM((CH_PER_TILE, CHUNK), jnp.int32),
            pltpu.VMEM((CH_PER_TILE, CHUNK), jnp.int32),
            [pltpu.VMEM((CHUNK, HID), jnp.float32) for _ in range(NBUF)],
            pltpu.VMEM((CHUNK, HID), jnp.float32),
            pltpu.VMEM_SHARED((NPAD, HID), jnp.float32),
            [pltpu.SemaphoreType.DMA for _ in range(NBUF)],
            [pltpu.SemaphoreType.DMA for _ in range(NBUF)],
        ],
        compiler_params=_SC_PARAMS,
    )
    def agg_kernel(g_hbm, src_hbm, dst_hbm, out_hbm, src_v, dst_v, rows,
                   zero_v, acc_sh, gsems, ssems):
        c = lax.axis_index("c")
        s = lax.axis_index("s")

        def fill_zero(i, carry):
            for k in range(HID // LANES):
                zero_v[i, pl.ds(k * LANES, LANES)] = jnp.zeros((LANES,), jnp.float32)
            return carry

        lax.fori_loop(0, CHUNK, fill_zero, 0)
        for z in range(ROWS_PER_TILE // CHUNK):
            pltpu.sync_copy(
                zero_v,
                acc_sh.at[pl.ds(s * ROWS_PER_TILE + z * CHUNK, CHUNK)])
        plsc.subcore_barrier()

        pltpu.sync_copy(src_hbm.at[c, s], src_v)
        pltpu.sync_copy(dst_hbm.at[c, s], dst_v)

        def body(i, carry):
            for k in range(NBUF):
                j = NBUF * i + k

                @pl.when(i > 0)
                def _(k=k, j=j):
                    pltpu.make_async_copy(
                        rows[k], acc_sh.at[dst_v.at[j]], ssems[k]).wait()

                pltpu.async_copy(g_hbm.at[src_v.at[j]], rows[k], gsems[k])
            for k in range(NBUF):
                j = NBUF * i + k
                pltpu.make_async_copy(
                    g_hbm.at[src_v.at[j]], rows[k], gsems[k]).wait()
                pltpu.async_copy(rows[k], acc_sh.at[dst_v.at[j]], ssems[k],
                                 add=True)
            return carry

        lax.fori_loop(0, CH_PER_TILE // NBUF, body, 0)
        for k in range(NBUF):
            pltpu.make_async_copy(rows[k], acc_sh.at[dst_v.at[0]],
                                  ssems[k]).wait()
        plsc.subcore_barrier()
        pltpu.sync_copy(
            acc_sh.at[pl.ds(s * ROWS_PER_TILE, ROWS_PER_TILE)],
            out_hbm.at[c, pl.ds(s * ROWS_PER_TILE, ROWS_PER_TILE)],
        )

    return agg_kernel(g, src_r, dst_r)


def _dinv_from(deg_ref):
    deg = deg_ref[0, :, 0] + deg_ref[1, :, 0]
    return lax.rsqrt(deg + 1.0)


def _first_body(x_ref, deg_ref, w1_ref, out_ref):
    dinv = _dinv_from(deg_ref)
    h = jnp.dot(x_ref[...], w1_ref[...], preferred_element_type=jnp.float32)
    out_ref[...] = h * dinv[:, None]


def _tc_first(x_pad, deg2, W1):
    return pl.pallas_call(
        _first_body,
        grid=(NPAD // BR,),
        in_specs=[
            pl.BlockSpec((BR, 128), lambda i: (i, 0)),
            pl.BlockSpec((NC, BR, DEGW), lambda i: (0, i, 0)),
            pl.BlockSpec((128, HID), lambda i: (0, 0)),
        ],
        out_specs=pl.BlockSpec((BR, HID), lambda i: (i, 0)),
        out_shape=jax.ShapeDtypeStruct((NPAD, HID), jnp.float32),
    )(x_pad, deg2, W1)


def _mid_body(p_ref, g1_ref, deg_ref, w2_ref, b1_ref, out_ref):
    dinv = _dinv_from(deg_ref)
    pp = p_ref[0] + p_ref[1] + g1_ref[...]
    t = jnp.maximum(pp * dinv[:, None] + b1_ref[...], 0.0)
    g2 = jnp.dot(t, w2_ref[...], preferred_element_type=jnp.float32) * dinv[:, None]
    row = pl.program_id(0) * BR + lax.broadcasted_iota(jnp.int32, (BR, 1), 0)
    out_ref[...] = jnp.where(row < NNODES, g2, 0.0)


def _tc_mid(p, g1, deg2, W2, b1r):
    return pl.pallas_call(
        _mid_body,
        grid=(NPAD // BR,),
        in_specs=[
            pl.BlockSpec((NC, BR, HID), lambda i: (0, i, 0)),
            pl.BlockSpec((BR, HID), lambda i: (i, 0)),
            pl.BlockSpec((NC, BR, DEGW), lambda i: (0, i, 0)),
            pl.BlockSpec((HID, HID), lambda i: (0, 0)),
            pl.BlockSpec((1, HID), lambda i: (0, 0)),
        ],
        out_specs=pl.BlockSpec((BR, HID), lambda i: (i, 0)),
        out_shape=jax.ShapeDtypeStruct((NPAD, HID), jnp.float32),
    )(p, g1, deg2, W2, b1r)


def _final_body(q_ref, g2_ref, deg_ref, b2_ref, w3_ref, b3_ref, out_ref):
    dinv = _dinv_from(deg_ref)
    qq = q_ref[0] + q_ref[1] + g2_ref[...]
    t = jnp.maximum(qq * dinv[:, None] + b2_ref[...], 0.0)
    o = jnp.sum(t * w3_ref[...], axis=1, keepdims=True) + b3_ref[0, 0]
    out_ref[...] = o


def _tc_final(q, g2, deg2, b2r, w3r, b3r):
    return pl.pallas_call(
        _final_body,
        grid=(NPAD // BR,),
        in_specs=[
            pl.BlockSpec((NC, BR, HID), lambda i: (0, i, 0)),
            pl.BlockSpec((BR, HID), lambda i: (i, 0)),
            pl.BlockSpec((NC, BR, DEGW), lambda i: (0, i, 0)),
            pl.BlockSpec((1, HID), lambda i: (0, 0)),
            pl.BlockSpec((1, HID), lambda i: (0, 0)),
            pl.BlockSpec((1, HID), lambda i: (0, 0)),
        ],
        out_specs=pl.BlockSpec((BR, 1), lambda i: (i, 0)),
        out_shape=jax.ShapeDtypeStruct((NPAD, 1), jnp.float32),
    )(q, g2, deg2, b2r, w3r, b3r)


def kernel(x, edge_index, W1, b1, W2, b2, W3, b3):
    n, _ = x.shape
    e = edge_index.shape[1]
    x_pad = jnp.pad(x, ((0, NPAD - n), (0, 0)))
    ep = jnp.pad(edge_index, ((0, 0), (0, EPAD - e)), constant_values=n)
    src_r = ep[0].reshape(NC, NS, CH_PER_TILE, CHUNK)
    dst_r = ep[1].reshape(NC, NS, CH_PER_TILE, CHUNK)

    deg2 = _sc_degree(dst_r)
    g1 = _tc_first(x_pad, deg2, W1)
    p = _sc_aggregate(g1, src_r, dst_r)
    g2 = _tc_mid(p, g1, deg2, W2, b1.reshape(1, HID))
    q = _sc_aggregate(g2, src_r, dst_r)
    res = _tc_final(
        q, g2, deg2,
        b2.reshape(1, HID),
        W3.reshape(1, HID),
        jnp.broadcast_to(b3.reshape(1, 1), (1, HID)),
    )
    return res[:n, 0]


# speedup vs baseline: 31.4334x; 1.7925x over previous
import functools

import jax
import jax.numpy as jnp
from jax import lax
from jax.experimental import pallas as pl
from jax.experimental.pallas import tpu as pltpu
from jax.experimental.pallas import tpu_sc as plsc

NC = 2
NS = 16
LANES = 16

NNODES = 10000
NPAD = 10240
HID = 64
CHUNK = 128
CH_PER_TILE = 80
EPAD = NC * NS * CH_PER_TILE * CHUNK
ROWS_PER_TILE = NPAD // NS
BR = 512
DEGW = 16
NBUF = 2


def _sc_mesh():
    return plsc.VectorSubcoreMesh(
        core_axis_name="c", subcore_axis_name="s", num_cores=NC, num_subcores=NS
    )


_SC_PARAMS = pltpu.CompilerParams(use_tc_tiling_on_sc=False)


def _sc_degree(dst_r):
    @functools.partial(
        pl.kernel,
        out_type=jax.ShapeDtypeStruct((NC, NPAD, DEGW), jnp.float32),
        mesh=_sc_mesh(),
        scratch_types=[
            pltpu.VMEM((CH_PER_TILE, CHUNK), jnp.int32),
            pltpu.VMEM((CHUNK, DEGW), jnp.float32),
            pltpu.VMEM((ROWS_PER_TILE, DEGW), jnp.float32),
            pltpu.VMEM_SHARED((NPAD, DEGW), jnp.float32),
            pltpu.SemaphoreType.DMA,
        ],
        compiler_params=_SC_PARAMS,
    )
    def deg_kernel(dst_hbm, out_hbm, dst_v, ones_v, zero_v, acc_sh, sem):
        c = lax.axis_index("c")
        s = lax.axis_index("s")

        def fill_ones(i, carry):
            ones_v[i, :] = jnp.ones((LANES,), jnp.float32)
            return carry

        lax.fori_loop(0, CHUNK, fill_ones, 0)

        def fill_zero(i, carry):
            zero_v[i, :] = jnp.zeros((LANES,), jnp.float32)
            return carry

        lax.fori_loop(0, ROWS_PER_TILE, fill_zero, 0)
        pltpu.sync_copy(zero_v, acc_sh.at[pl.ds(s * ROWS_PER_TILE, ROWS_PER_TILE)])
        plsc.subcore_barrier()

        pltpu.sync_copy(dst_hbm.at[c, s], dst_v)

        def body(j, carry):
            for k in range(8):
                pltpu.async_copy(ones_v, acc_sh.at[dst_v.at[j * 8 + k]], sem,
                                 add=True)
            for k in range(8):
                pltpu.make_async_copy(ones_v, acc_sh.at[dst_v.at[j * 8 + k]],
                                      sem).wait()
            return carry

        lax.fori_loop(0, CH_PER_TILE // 8, body, 0)
        plsc.subcore_barrier()
        pltpu.sync_copy(
            acc_sh.at[pl.ds(s * ROWS_PER_TILE, ROWS_PER_TILE)],
            out_hbm.at[c, pl.ds(s * ROWS_PER_TILE, ROWS_PER_TILE)],
        )

    return deg_kernel(dst_r)


def _sc_aggregate(g, src_r, dst_r):
    @functools.partial(
        pl.kernel,
        out_type=jax.ShapeDtypeStruct((NC, NPAD, HID), jnp.float32),
        mesh=_sc_mesh(),
        scratch_types=[
            pltpu.VMEM((CH_PER_TILE, CHUNK), jnp.int32),
            pltpu.VMEM((CH_PER_TILE, CHUNK), jnp.int32),
            [pltpu.VMEM((CHUNK, HID), jnp.float32) for _ in range(NBUF)],
            pltpu.VMEM((CHUNK, HID), jnp.float32),
            pltpu.VMEM_SHARED((NPAD, HID), jnp.float32),
            pltpu.VMEM_SHARED((NPAD, HID), jnp.float32),
            [pltpu.SemaphoreType.DMA for _ in range(NBUF)],
            [pltpu.SemaphoreType.DMA for _ in range(NBUF)],
        ],
        compiler_params=_SC_PARAMS,
    )
    def agg_kernel(g_hbm, src_hbm, dst_hbm, out_hbm, src_v, dst_v, rows,
                   zero_v, acc_sh, g_sh, gsems, ssems):
        c = lax.axis_index("c")
        s = lax.axis_index("s")

        def fill_zero(i, carry):
            for k in range(HID // LANES):
                zero_v[i, pl.ds(k * LANES, LANES)] = jnp.zeros((LANES,), jnp.float32)
            return carry

        lax.fori_loop(0, CHUNK, fill_zero, 0)
        for z in range(ROWS_PER_TILE // CHUNK):
            pltpu.sync_copy(
                zero_v,
                acc_sh.at[pl.ds(s * ROWS_PER_TILE + z * CHUNK, CHUNK)])
        pltpu.sync_copy(g_hbm.at[pl.ds(s * ROWS_PER_TILE, ROWS_PER_TILE)],
                        g_sh.at[pl.ds(s * ROWS_PER_TILE, ROWS_PER_TILE)])
        plsc.subcore_barrier()

        pltpu.sync_copy(src_hbm.at[c, s], src_v)
        pltpu.sync_copy(dst_hbm.at[c, s], dst_v)

        def body(i, carry):
            for k in range(NBUF):
                j = NBUF * i + k

                @pl.when(i > 0)
                def _(k=k, j=j):
                    pltpu.make_async_copy(
                        rows[k], acc_sh.at[dst_v.at[j]], ssems[k]).wait()

                pltpu.async_copy(g_sh.at[src_v.at[j]], rows[k], gsems[k])
            for k in range(NBUF):
                j = NBUF * i + k
                pltpu.make_async_copy(
                    g_sh.at[src_v.at[j]], rows[k], gsems[k]).wait()
                pltpu.async_copy(rows[k], acc_sh.at[dst_v.at[j]], ssems[k],
                                 add=True)
            return carry

        lax.fori_loop(0, CH_PER_TILE // NBUF, body, 0)
        for k in range(NBUF):
            pltpu.make_async_copy(rows[k], acc_sh.at[dst_v.at[0]],
                                  ssems[k]).wait()
        plsc.subcore_barrier()
        pltpu.sync_copy(
            acc_sh.at[pl.ds(s * ROWS_PER_TILE, ROWS_PER_TILE)],
            out_hbm.at[c, pl.ds(s * ROWS_PER_TILE, ROWS_PER_TILE)],
        )

    return agg_kernel(g, src_r, dst_r)


def _dinv_from(deg_ref):
    deg = deg_ref[0, :, 0] + deg_ref[1, :, 0]
    return lax.rsqrt(deg + 1.0)


def _first_body(x_ref, deg_ref, w1_ref, out_ref):
    dinv = _dinv_from(deg_ref)
    h = jnp.dot(x_ref[...], w1_ref[...], preferred_element_type=jnp.float32)
    out_ref[...] = h * dinv[:, None]


def _tc_first(x_pad, deg2, W1):
    return pl.pallas_call(
        _first_body,
        grid=(NPAD // BR,),
        in_specs=[
            pl.BlockSpec((BR, 128), lambda i: (i, 0)),
            pl.BlockSpec((NC, BR, DEGW), lambda i: (0, i, 0)),
            pl.BlockSpec((128, HID), lambda i: (0, 0)),
        ],
        out_specs=pl.BlockSpec((BR, HID), lambda i: (i, 0)),
        out_shape=jax.ShapeDtypeStruct((NPAD, HID), jnp.float32),
    )(x_pad, deg2, W1)


def _mid_body(p_ref, g1_ref, deg_ref, w2_ref, b1_ref, out_ref):
    dinv = _dinv_from(deg_ref)
    pp = p_ref[0] + p_ref[1] + g1_ref[...]
    t = jnp.maximum(pp * dinv[:, None] + b1_ref[...], 0.0)
    g2 = jnp.dot(t, w2_ref[...], preferred_element_type=jnp.float32) * dinv[:, None]
    row = pl.program_id(0) * BR + lax.broadcasted_iota(jnp.int32, (BR, 1), 0)
    out_ref[...] = jnp.where(row < NNODES, g2, 0.0)


def _tc_mid(p, g1, deg2, W2, b1r):
    return pl.pallas_call(
        _mid_body,
        grid=(NPAD // BR,),
        in_specs=[
            pl.BlockSpec((NC, BR, HID), lambda i: (0, i, 0)),
            pl.BlockSpec((BR, HID), lambda i: (i, 0)),
            pl.BlockSpec((NC, BR, DEGW), lambda i: (0, i, 0)),
            pl.BlockSpec((HID, HID), lambda i: (0, 0)),
            pl.BlockSpec((1, HID), lambda i: (0, 0)),
        ],
        out_specs=pl.BlockSpec((BR, HID), lambda i: (i, 0)),
        out_shape=jax.ShapeDtypeStruct((NPAD, HID), jnp.float32),
    )(p, g1, deg2, W2, b1r)


def _final_body(q_ref, g2_ref, deg_ref, b2_ref, w3_ref, b3_ref, out_ref):
    dinv = _dinv_from(deg_ref)
    qq = q_ref[0] + q_ref[1] + g2_ref[...]
    t = jnp.maximum(qq * dinv[:, None] + b2_ref[...], 0.0)
    o = jnp.sum(t * w3_ref[...], axis=1, keepdims=True) + b3_ref[0, 0]
    out_ref[...] = o


def _tc_final(q, g2, deg2, b2r, w3r, b3r):
    return pl.pallas_call(
        _final_body,
        grid=(NPAD // BR,),
        in_specs=[
            pl.BlockSpec((NC, BR, HID), lambda i: (0, i, 0)),
            pl.BlockSpec((BR, HID), lambda i: (i, 0)),
            pl.BlockSpec((NC, BR, DEGW), lambda i: (0, i, 0)),
            pl.BlockSpec((1, HID), lambda i: (0, 0)),
            pl.BlockSpec((1, HID), lambda i: (0, 0)),
            pl.BlockSpec((1, HID), lambda i: (0, 0)),
        ],
        out_specs=pl.BlockSpec((BR, 1), lambda i: (i, 0)),
        out_shape=jax.ShapeDtypeStruct((NPAD, 1), jnp.float32),
    )(q, g2, deg2, b2r, w3r, b3r)


def kernel(x, edge_index, W1, b1, W2, b2, W3, b3):
    n, _ = x.shape
    e = edge_index.shape[1]
    x_pad = jnp.pad(x, ((0, NPAD - n), (0, 0)))
    ep = jnp.pad(edge_index, ((0, 0), (0, EPAD - e)), constant_values=n)
    src_r = ep[0].reshape(NC, NS, CH_PER_TILE, CHUNK)
    dst_r = ep[1].reshape(NC, NS, CH_PER_TILE, CHUNK)

    deg2 = _sc_degree(dst_r)
    g1 = _tc_first(x_pad, deg2, W1)
    p = _sc_aggregate(g1, src_r, dst_r)
    g2 = _tc_mid(p, g1, deg2, W2, b1.reshape(1, HID))
    q = _sc_aggregate(g2, src_r, dst_r)
    res = _tc_final(
        q, g2, deg2,
        b2.reshape(1, HID),
        W3.reshape(1, HID),
        jnp.broadcast_to(b3.reshape(1, 1), (1, HID)),
    )
    return res[:n, 0]


# speedup vs baseline: 32.9998x; 1.0498x over previous
import functools

import jax
import jax.numpy as jnp
from jax import lax
from jax.experimental import pallas as pl
from jax.experimental.pallas import tpu as pltpu
from jax.experimental.pallas import tpu_sc as plsc

NC = 2
NS = 16
LANES = 16

NNODES = 10000
NPAD = 10240
HID = 64
CHUNK = 128
CH_PER_TILE = 80
EPAD = NC * NS * CH_PER_TILE * CHUNK
ROWS_PER_TILE = NPAD // NS
BR = 1024
DEGW = 8
NBUF = 2


def _sc_mesh():
    return plsc.VectorSubcoreMesh(
        core_axis_name="c", subcore_axis_name="s", num_cores=NC, num_subcores=NS
    )


_SC_PARAMS = pltpu.CompilerParams(use_tc_tiling_on_sc=False)


def _sc_degree(dst_r, const8):
    @functools.partial(
        pl.kernel,
        out_type=jax.ShapeDtypeStruct((NC, NPAD, DEGW), jnp.float32),
        mesh=_sc_mesh(),
        scratch_types=[
            pltpu.VMEM((CH_PER_TILE, CHUNK), jnp.int32),
            pltpu.VMEM((CHUNK, DEGW), jnp.float32),
            pltpu.VMEM_SHARED((NPAD, DEGW), jnp.float32),
            pltpu.SemaphoreType.DMA,
        ],
        compiler_params=_SC_PARAMS,
    )
    def deg_kernel(dst_hbm, c8_hbm, out_hbm, dst_v, ones_v, acc_sh, sem):
        c = lax.axis_index("c")
        s = lax.axis_index("s")
        pltpu.sync_copy(c8_hbm.at[0], ones_v)
        for z in range(ROWS_PER_TILE // CHUNK):
            pltpu.sync_copy(
                c8_hbm.at[1],
                acc_sh.at[pl.ds(s * ROWS_PER_TILE + z * CHUNK, CHUNK)])
        plsc.subcore_barrier()

        pltpu.sync_copy(dst_hbm.at[c, s], dst_v)

        def body(j, carry):
            for k in range(8):
                pltpu.async_copy(ones_v, acc_sh.at[dst_v.at[j * 8 + k]], sem,
                                 add=True)
            for k in range(8):
                pltpu.make_async_copy(ones_v, acc_sh.at[dst_v.at[j * 8 + k]],
                                      sem).wait()
            return carry

        lax.fori_loop(0, CH_PER_TILE // 8, body, 0)
        plsc.subcore_barrier()
        pltpu.sync_copy(
            acc_sh.at[pl.ds(s * ROWS_PER_TILE, ROWS_PER_TILE)],
            out_hbm.at[c, pl.ds(s * ROWS_PER_TILE, ROWS_PER_TILE)],
        )

    return deg_kernel(dst_r, const8)


def _sc_aggregate(g, src_r, dst_r):
    @functools.partial(
        pl.kernel,
        out_type=jax.ShapeDtypeStruct((NC, NPAD, HID), jnp.float32),
        mesh=_sc_mesh(),
        scratch_types=[
            pltpu.VMEM((CH_PER_TILE, CHUNK), jnp.int32),
            pltpu.VMEM((CH_PER_TILE, CHUNK), jnp.int32),
            [pltpu.VMEM((CHUNK, HID), jnp.float32) for _ in range(NBUF)],
            pltpu.VMEM((CHUNK, HID), jnp.float32),
            pltpu.VMEM_SHARED((NPAD, HID), jnp.float32),
            pltpu.VMEM_SHARED((NPAD, HID), jnp.float32),
            [pltpu.SemaphoreType.DMA for _ in range(NBUF)],
            [pltpu.SemaphoreType.DMA for _ in range(NBUF)],
        ],
        compiler_params=_SC_PARAMS,
    )
    def agg_kernel(g_hbm, src_hbm, dst_hbm, out_hbm, src_v, dst_v, rows,
                   zero_v, acc_sh, g_sh, gsems, ssems):
        c = lax.axis_index("c")
        s = lax.axis_index("s")

        def fill_zero(i, carry):
            for k in range(HID // LANES):
                zero_v[i, pl.ds(k * LANES, LANES)] = jnp.zeros((LANES,), jnp.float32)
            return carry

        lax.fori_loop(0, CHUNK, fill_zero, 0)
        for z in range(ROWS_PER_TILE // CHUNK):
            pltpu.sync_copy(
                zero_v,
                acc_sh.at[pl.ds(s * ROWS_PER_TILE + z * CHUNK, CHUNK)])
        pltpu.sync_copy(g_hbm.at[pl.ds(s * ROWS_PER_TILE, ROWS_PER_TILE)],
                        g_sh.at[pl.ds(s * ROWS_PER_TILE, ROWS_PER_TILE)])
        plsc.subcore_barrier()

        pltpu.sync_copy(src_hbm.at[c, s], src_v)
        pltpu.sync_copy(dst_hbm.at[c, s], dst_v)

        def body(i, carry):
            for k in range(NBUF):
                j = NBUF * i + k

                @pl.when(i > 0)
                def _(k=k, j=j):
                    pltpu.make_async_copy(
                        rows[k], acc_sh.at[dst_v.at[j]], ssems[k]).wait()

                pltpu.async_copy(g_sh.at[src_v.at[j]], rows[k], gsems[k])
            for k in range(NBUF):
                j = NBUF * i + k
                pltpu.make_async_copy(
                    g_sh.at[src_v.at[j]], rows[k], gsems[k]).wait()
                pltpu.async_copy(rows[k], acc_sh.at[dst_v.at[j]], ssems[k],
                                 add=True)
            return carry

        lax.fori_loop(0, CH_PER_TILE // NBUF, body, 0)
        for k in range(NBUF):
            pltpu.make_async_copy(rows[k], acc_sh.at[dst_v.at[0]],
                                  ssems[k]).wait()
        plsc.subcore_barrier()
        pltpu.sync_copy(
            acc_sh.at[pl.ds(s * ROWS_PER_TILE, ROWS_PER_TILE)],
            out_hbm.at[c, pl.ds(s * ROWS_PER_TILE, ROWS_PER_TILE)],
        )

    return agg_kernel(g, src_r, dst_r)


def _dinv_from(deg_ref):
    deg = deg_ref[0, :, 0] + deg_ref[1, :, 0]
    return lax.rsqrt(deg + 1.0)


def _first_body(x_ref, deg_ref, w1_ref, out_ref):
    dinv = _dinv_from(deg_ref)
    h = jnp.dot(x_ref[...], w1_ref[...], preferred_element_type=jnp.float32)
    out_ref[...] = h * dinv[:, None]


def _tc_first(x_pad, deg2, W1):
    return pl.pallas_call(
        _first_body,
        grid=(NPAD // BR,),
        in_specs=[
            pl.BlockSpec((BR, 128), lambda i: (i, 0)),
            pl.BlockSpec((NC, BR, DEGW), lambda i: (0, i, 0)),
            pl.BlockSpec((128, HID), lambda i: (0, 0)),
        ],
        out_specs=pl.BlockSpec((BR, HID), lambda i: (i, 0)),
        out_shape=jax.ShapeDtypeStruct((NPAD, HID), jnp.float32),
    )(x_pad, deg2, W1)


def _mid_body(p_ref, g1_ref, deg_ref, w2_ref, b1_ref, out_ref):
    dinv = _dinv_from(deg_ref)
    pp = p_ref[0] + p_ref[1] + g1_ref[...]
    t = jnp.maximum(pp * dinv[:, None] + b1_ref[...], 0.0)
    g2 = jnp.dot(t, w2_ref[...], preferred_element_type=jnp.float32) * dinv[:, None]
    row = pl.program_id(0) * BR + lax.broadcasted_iota(jnp.int32, (BR, 1), 0)
    out_ref[...] = jnp.where(row < NNODES, g2, 0.0)


def _tc_mid(p, g1, deg2, W2, b1r):
    return pl.pallas_call(
        _mid_body,
        grid=(NPAD // BR,),
        in_specs=[
            pl.BlockSpec((NC, BR, HID), lambda i: (0, i, 0)),
            pl.BlockSpec((BR, HID), lambda i: (i, 0)),
            pl.BlockSpec((NC, BR, DEGW), lambda i: (0, i, 0)),
            pl.BlockSpec((HID, HID), lambda i: (0, 0)),
            pl.BlockSpec((1, HID), lambda i: (0, 0)),
        ],
        out_specs=pl.BlockSpec((BR, HID), lambda i: (i, 0)),
        out_shape=jax.ShapeDtypeStruct((NPAD, HID), jnp.float32),
    )(p, g1, deg2, W2, b1r)


def _final_body(q_ref, g2_ref, deg_ref, b2_ref, w3_ref, b3_ref, out_ref):
    dinv = _dinv_from(deg_ref)
    qq = q_ref[0] + q_ref[1] + g2_ref[...]
    t = jnp.maximum(qq * dinv[:, None] + b2_ref[...], 0.0)
    o = jnp.sum(t * w3_ref[...], axis=1, keepdims=True) + b3_ref[0, 0]
    out_ref[...] = o


def _tc_final(q, g2, deg2, b2r, w3r, b3r):
    return pl.pallas_call(
        _final_body,
        grid=(NPAD // BR,),
        in_specs=[
            pl.BlockSpec((NC, BR, HID), lambda i: (0, i, 0)),
            pl.BlockSpec((BR, HID), lambda i: (i, 0)),
            pl.BlockSpec((NC, BR, DEGW), lambda i: (0, i, 0)),
            pl.BlockSpec((1, HID), lambda i: (0, 0)),
            pl.BlockSpec((1, HID), lambda i: (0, 0)),
            pl.BlockSpec((1, HID), lambda i: (0, 0)),
        ],
        out_specs=pl.BlockSpec((BR, 1), lambda i: (i, 0)),
        out_shape=jax.ShapeDtypeStruct((NNODES, 1), jnp.float32),
    )(q, g2, deg2, b2r, w3r, b3r)


def kernel(x, edge_index, W1, b1, W2, b2, W3, b3):
    n, _ = x.shape
    e = edge_index.shape[1]
    ep = jnp.pad(edge_index, ((0, 0), (0, EPAD - e)), constant_values=n)
    src_r = ep[0].reshape(NC, NS, CH_PER_TILE, CHUNK)
    dst_r = ep[1].reshape(NC, NS, CH_PER_TILE, CHUNK)

    const8 = jnp.stack([jnp.ones((CHUNK, DEGW), jnp.float32),
                        jnp.zeros((CHUNK, DEGW), jnp.float32)])
    deg2 = _sc_degree(dst_r, const8)
    g1 = _tc_first(x, deg2, W1)
    p = _sc_aggregate(g1, src_r, dst_r)
    g2 = _tc_mid(p, g1, deg2, W2, b1.reshape(1, HID))
    q = _sc_aggregate(g2, src_r, dst_r)
    res = _tc_final(
        q, g2, deg2,
        b2.reshape(1, HID),
        W3.reshape(1, HID),
        jnp.broadcast_to(b3.reshape(1, 1), (1, HID)),
    )
    return res[:, 0]


# speedup vs baseline: 36.4776x; 1.1054x over previous
import functools

import jax
import jax.numpy as jnp
from jax import lax
from jax.experimental import pallas as pl
from jax.experimental.pallas import tpu as pltpu
from jax.experimental.pallas import tpu_sc as plsc

NC = 2
NS = 16
LANES = 16

NNODES = 10000
NPAD = 10240
HID = 64
CHUNK = 128
CH_PER_TILE = 80
EPAD = NC * NS * CH_PER_TILE * CHUNK
ROWS_PER_TILE = NPAD // NS
BR = 1024
DEGW = 8
NBUF = 2


def _sc_mesh():
    return plsc.VectorSubcoreMesh(
        core_axis_name="c", subcore_axis_name="s", num_cores=NC, num_subcores=NS
    )


_SC_PARAMS = pltpu.CompilerParams(use_tc_tiling_on_sc=False)


def _sc_degree(dst_r, const8):
    @functools.partial(
        pl.kernel,
        out_type=jax.ShapeDtypeStruct((NC, NPAD, DEGW), jnp.float32),
        mesh=_sc_mesh(),
        scratch_types=[
            pltpu.VMEM((CH_PER_TILE, CHUNK), jnp.int32),
            pltpu.VMEM((CHUNK, DEGW), jnp.float32),
            pltpu.VMEM_SHARED((NPAD, DEGW), jnp.float32),
            pltpu.SemaphoreType.DMA,
        ],
        compiler_params=_SC_PARAMS,
    )
    def deg_kernel(dst_hbm, c8_hbm, out_hbm, dst_v, ones_v, acc_sh, sem):
        c = lax.axis_index("c")
        s = lax.axis_index("s")
        pltpu.sync_copy(c8_hbm.at[0], ones_v)
        for z in range(ROWS_PER_TILE // CHUNK):
            pltpu.sync_copy(
                c8_hbm.at[1],
                acc_sh.at[pl.ds(s * ROWS_PER_TILE + z * CHUNK, CHUNK)])
        plsc.subcore_barrier()

        pltpu.sync_copy(dst_hbm.at[c, s], dst_v)

        def body(j, carry):
            for k in range(8):
                pltpu.async_copy(ones_v, acc_sh.at[dst_v.at[j * 8 + k]], sem,
                                 add=True)
            for k in range(8):
                pltpu.make_async_copy(ones_v, acc_sh.at[dst_v.at[j * 8 + k]],
                                      sem).wait()
            return carry

        lax.fori_loop(0, CH_PER_TILE // 8, body, 0)
        plsc.subcore_barrier()
        pltpu.sync_copy(
            acc_sh.at[pl.ds(s * ROWS_PER_TILE, ROWS_PER_TILE)],
            out_hbm.at[c, pl.ds(s * ROWS_PER_TILE, ROWS_PER_TILE)],
        )

    return deg_kernel(dst_r, const8)


def _sc_aggregate(g, src_r, dst_r):
    @functools.partial(
        pl.kernel,
        out_type=jax.ShapeDtypeStruct((NC, NPAD, HID), jnp.float32),
        mesh=_sc_mesh(),
        scratch_types=[
            pltpu.VMEM((CH_PER_TILE, CHUNK), jnp.int32),
            pltpu.VMEM((CH_PER_TILE, CHUNK), jnp.int32),
            [pltpu.VMEM((CHUNK, HID), jnp.float32) for _ in range(NBUF)],
            pltpu.VMEM((CHUNK, HID), jnp.float32),
            pltpu.VMEM_SHARED((NPAD, HID), jnp.float32),
            pltpu.VMEM_SHARED((NPAD, HID), jnp.float32),
            [pltpu.SemaphoreType.DMA for _ in range(NBUF)],
            [pltpu.SemaphoreType.DMA for _ in range(NBUF)],
        ],
        compiler_params=_SC_PARAMS,
    )
    def agg_kernel(g_hbm, src_hbm, dst_hbm, out_hbm, src_v, dst_v, rows,
                   zero_v, acc_sh, g_sh, gsems, ssems):
        c = lax.axis_index("c")
        s = lax.axis_index("s")

        def fill_zero(i, carry):
            for k in range(HID // LANES):
                zero_v[i, pl.ds(k * LANES, LANES)] = jnp.zeros((LANES,), jnp.float32)
            return carry

        lax.fori_loop(0, CHUNK, fill_zero, 0)
        for z in range(ROWS_PER_TILE // CHUNK):
            pltpu.sync_copy(
                zero_v,
                acc_sh.at[pl.ds(s * ROWS_PER_TILE + z * CHUNK, CHUNK)])
        pltpu.sync_copy(g_hbm.at[pl.ds(s * ROWS_PER_TILE, ROWS_PER_TILE)],
                        g_sh.at[pl.ds(s * ROWS_PER_TILE, ROWS_PER_TILE)])
        plsc.subcore_barrier()

        pltpu.sync_copy(src_hbm.at[c, s], src_v)
        pltpu.sync_copy(dst_hbm.at[c, s], dst_v)

        def body(i, carry):
            for k in range(NBUF):
                j = NBUF * i + k

                @pl.when(i > 0)
                def _(k=k, j=j):
                    pltpu.make_async_copy(
                        rows[k], acc_sh.at[dst_v.at[j]], ssems[k]).wait()

                pltpu.async_copy(g_sh.at[src_v.at[j]], rows[k], gsems[k])
            for k in range(NBUF):
                j = NBUF * i + k
                pltpu.make_async_copy(
                    g_sh.at[src_v.at[j]], rows[k], gsems[k]).wait()
                pltpu.async_copy(rows[k], acc_sh.at[dst_v.at[j]], ssems[k],
                                 add=True)
            return carry

        lax.fori_loop(0, CH_PER_TILE // NBUF, body, 0)
        for k in range(NBUF):
            pltpu.make_async_copy(rows[k], acc_sh.at[dst_v.at[0]],
                                  ssems[k]).wait()
        plsc.subcore_barrier()
        pltpu.sync_copy(
            acc_sh.at[pl.ds(s * ROWS_PER_TILE, ROWS_PER_TILE)],
            out_hbm.at[c, pl.ds(s * ROWS_PER_TILE, ROWS_PER_TILE)],
        )

    return agg_kernel(g, src_r, dst_r)


def _dot1(a, b):
    return jnp.dot(a.astype(jnp.bfloat16), b.astype(jnp.bfloat16),
                   preferred_element_type=jnp.float32)


def _dinv_from(deg_ref):
    deg = deg_ref[0, :, 0] + deg_ref[1, :, 0]
    return 1.0 / jnp.sqrt(deg + 1.0)


def _first_body(x_ref, deg_ref, w1_ref, g1_ref, db_ref):
    dinv = _dinv_from(deg_ref)
    h = _dot1(x_ref[...], w1_ref[...])
    g1_ref[...] = h * dinv[:, None]
    db_ref[...] = jnp.broadcast_to(dinv[:, None], (BR, HID))


def _tc_first(x, deg2, W1):
    return pl.pallas_call(
        _first_body,
        grid=(NPAD // BR,),
        in_specs=[
            pl.BlockSpec((BR, 128), lambda i: (i, 0)),
            pl.BlockSpec((NC, BR, DEGW), lambda i: (0, i, 0)),
            pl.BlockSpec((128, HID), lambda i: (0, 0)),
        ],
        out_specs=[
            pl.BlockSpec((BR, HID), lambda i: (i, 0)),
            pl.BlockSpec((BR, HID), lambda i: (i, 0)),
        ],
        out_shape=[
            jax.ShapeDtypeStruct((NPAD, HID), jnp.float32),
            jax.ShapeDtypeStruct((NPAD, HID), jnp.float32),
        ],
    )(x, deg2, W1)


NH = NPAD // 2
BRP = BR // 2


def _mid_body(p_ref, g1_ref, dp_ref, w22_ref, b1_ref, out_ref):
    dp = dp_ref[...]
    pp = p_ref[0] + p_ref[1] + g1_ref[...]
    t = jnp.maximum(pp * dp + b1_ref[...], 0.0)
    g2 = _dot1(t, w22_ref[...]) * dp
    row = pl.program_id(0) * BRP + lax.broadcasted_iota(jnp.int32, (BRP, 1), 0)
    out_ref[...] = jnp.where(row < NNODES // 2, g2, 0.0)


def _tc_mid(p128, g1p, dinvp, W22, b1p):
    return pl.pallas_call(
        _mid_body,
        grid=(NH // BRP,),
        in_specs=[
            pl.BlockSpec((NC, BRP, 128), lambda i: (0, i, 0)),
            pl.BlockSpec((BRP, 128), lambda i: (i, 0)),
            pl.BlockSpec((BRP, 128), lambda i: (i, 0)),
            pl.BlockSpec((128, 128), lambda i: (0, 0)),
            pl.BlockSpec((1, 128), lambda i: (0, 0)),
        ],
        out_specs=pl.BlockSpec((BRP, 128), lambda i: (i, 0)),
        out_shape=jax.ShapeDtypeStruct((NH, 128), jnp.float32),
    )(p128, g1p, dinvp, W22, b1p)


def _final_body(q_ref, g2_ref, dp_ref, b2_ref, w3_ref, b3_ref, out_ref):
    qq = q_ref[0] + q_ref[1] + g2_ref[...]
    t = jnp.maximum(qq * dp_ref[...] + b2_ref[...], 0.0)
    tb = t.astype(jnp.bfloat16).astype(jnp.float32)
    wb = w3_ref[...].astype(jnp.bfloat16).astype(jnp.float32)
    m = tb * wb
    o_lo = jnp.sum(m[:, :HID], axis=1, keepdims=True)
    o_hi = jnp.sum(m[:, HID:], axis=1, keepdims=True)
    out_ref[...] = jnp.concatenate([o_lo, o_hi], axis=1) + b3_ref[0, 0]


def _tc_final(q128, g2p, dinvp, b2p, w3p, b3r):
    return pl.pallas_call(
        _final_body,
        grid=(NH // BRP,),
        in_specs=[
            pl.BlockSpec((NC, BRP, 128), lambda i: (0, i, 0)),
            pl.BlockSpec((BRP, 128), lambda i: (i, 0)),
            pl.BlockSpec((BRP, 128), lambda i: (i, 0)),
            pl.BlockSpec((1, 128), lambda i: (0, 0)),
            pl.BlockSpec((1, 128), lambda i: (0, 0)),
            pl.BlockSpec((1, 128), lambda i: (0, 0)),
        ],
        out_specs=pl.BlockSpec((BRP, 2), lambda i: (i, 0)),
        out_shape=jax.ShapeDtypeStruct((NH, 2), jnp.float32),
    )(q128, g2p, dinvp, b2p, w3p, b3r)


def kernel(x, edge_index, W1, b1, W2, b2, W3, b3):
    n, _ = x.shape
    e = edge_index.shape[1]
    ep = jnp.pad(edge_index, ((0, 0), (0, EPAD - e)), constant_values=n)
    src_r = ep[0].reshape(NC, NS, CH_PER_TILE, CHUNK)
    dst_r = ep[1].reshape(NC, NS, CH_PER_TILE, CHUNK)

    const8 = jnp.stack([jnp.ones((CHUNK, DEGW), jnp.float32),
                        jnp.zeros((CHUNK, DEGW), jnp.float32)])
    w3row = W3.reshape(1, HID)
    w3p = jnp.concatenate([w3row, w3row], axis=1)
    b1p = jnp.concatenate([b1, b1]).reshape(1, 128)
    b2p = jnp.concatenate([b2, b2]).reshape(1, 128)
    b3r = jnp.broadcast_to(b3.reshape(1, 1), (1, 128))
    W22 = jnp.zeros((128, 128), jnp.float32)
    W22 = W22.at[:HID, :HID].set(W2).at[HID:, HID:].set(W2)

    deg2 = _sc_degree(dst_r, const8)
    g1, dinvb = _tc_first(x, deg2, W1)
    dinvp = dinvb.reshape(NH, 128)
    p = _sc_aggregate(g1, src_r, dst_r)
    g2p = _tc_mid(p.reshape(NC, NH, 128), g1.reshape(NH, 128), dinvp, W22, b1p)
    q = _sc_aggregate(g2p.reshape(NPAD, HID), src_r, dst_r)
    res = _tc_final(q.reshape(NC, NH, 128), g2p, dinvp, b2p, w3p, b3r)
    return res.reshape(-1)[:n]


# speedup vs baseline: 37.9160x; 1.0394x over previous
import functools

import jax
import jax.numpy as jnp
from jax import lax
from jax.experimental import pallas as pl
from jax.experimental.pallas import tpu as pltpu
from jax.experimental.pallas import tpu_sc as plsc

NC = 2
NS = 16
LANES = 16

NNODES = 10000
NPAD = 10240
HID = 64
CHUNK = 128
CH_PER_TILE = 80
EPAD = NC * NS * CH_PER_TILE * CHUNK
ROWS_PER_TILE = NPAD // NS
BR = 1024
DEGW = 8
NBUF = 2


def _sc_mesh():
    return plsc.VectorSubcoreMesh(
        core_axis_name="c", subcore_axis_name="s", num_cores=NC, num_subcores=NS
    )


_SC_PARAMS = pltpu.CompilerParams(use_tc_tiling_on_sc=False)


def _sc_degree(dst_r, const8):
    @functools.partial(
        pl.kernel,
        out_type=jax.ShapeDtypeStruct((NC, NPAD, DEGW), jnp.float32),
        mesh=_sc_mesh(),
        scratch_types=[
            pltpu.VMEM((CH_PER_TILE, CHUNK), jnp.int32),
            pltpu.VMEM((CHUNK, DEGW), jnp.float32),
            pltpu.VMEM_SHARED((NPAD, DEGW), jnp.float32),
            pltpu.SemaphoreType.DMA,
        ],
        compiler_params=_SC_PARAMS,
    )
    def deg_kernel(dst_hbm, c8_hbm, out_hbm, dst_v, ones_v, acc_sh, sem):
        c = lax.axis_index("c")
        s = lax.axis_index("s")
        pltpu.sync_copy(c8_hbm.at[0], ones_v)
        for z in range(ROWS_PER_TILE // CHUNK):
            pltpu.sync_copy(
                c8_hbm.at[1],
                acc_sh.at[pl.ds(s * ROWS_PER_TILE + z * CHUNK, CHUNK)])
        plsc.subcore_barrier()

        pltpu.sync_copy(dst_hbm.at[c, s], dst_v)

        def body(j, carry):
            for k in range(8):
                pltpu.async_copy(ones_v, acc_sh.at[dst_v.at[j * 8 + k]], sem,
                                 add=True)
            for k in range(8):
                pltpu.make_async_copy(ones_v, acc_sh.at[dst_v.at[j * 8 + k]],
                                      sem).wait()
            return carry

        lax.fori_loop(0, CH_PER_TILE // 8, body, 0)
        plsc.subcore_barrier()
        pltpu.sync_copy(
            acc_sh.at[pl.ds(s * ROWS_PER_TILE, ROWS_PER_TILE)],
            out_hbm.at[c, pl.ds(s * ROWS_PER_TILE, ROWS_PER_TILE)],
        )

    return deg_kernel(dst_r, const8)


def _sc_aggregate(g, src_r, dst_r):
    @functools.partial(
        pl.kernel,
        out_type=jax.ShapeDtypeStruct((NC, NPAD, HID), jnp.float32),
        mesh=_sc_mesh(),
        scratch_types=[
            pltpu.VMEM((CH_PER_TILE, CHUNK), jnp.int32),
            pltpu.VMEM((CH_PER_TILE, CHUNK), jnp.int32),
            [pltpu.VMEM((CHUNK, HID), jnp.float32) for _ in range(NBUF)],
            pltpu.VMEM((CHUNK, HID), jnp.float32),
            pltpu.VMEM_SHARED((NPAD, HID), jnp.float32),
            pltpu.VMEM_SHARED((NPAD, HID), jnp.float32),
            [pltpu.SemaphoreType.DMA for _ in range(NBUF)],
            [pltpu.SemaphoreType.DMA for _ in range(NBUF)],
        ],
        compiler_params=_SC_PARAMS,
    )
    def agg_kernel(g_hbm, src_hbm, dst_hbm, out_hbm, src_v, dst_v, rows,
                   zero_v, acc_sh, g_sh, gsems, ssems):
        c = lax.axis_index("c")
        s = lax.axis_index("s")

        def fill_zero(i, carry):
            for k in range(HID // LANES):
                zero_v[i, pl.ds(k * LANES, LANES)] = jnp.zeros((LANES,), jnp.float32)
            return carry

        lax.fori_loop(0, CHUNK, fill_zero, 0)
        for z in range(ROWS_PER_TILE // CHUNK):
            pltpu.sync_copy(
                zero_v,
                acc_sh.at[pl.ds(s * ROWS_PER_TILE + z * CHUNK, CHUNK)])
        pltpu.sync_copy(g_hbm.at[pl.ds(s * ROWS_PER_TILE, ROWS_PER_TILE)],
                        g_sh.at[pl.ds(s * ROWS_PER_TILE, ROWS_PER_TILE)])
        plsc.subcore_barrier()

        pltpu.sync_copy(src_hbm.at[c, s], src_v)
        pltpu.sync_copy(dst_hbm.at[c, s], dst_v)

        def body(i, carry):
            for k in range(NBUF):
                j = NBUF * i + k

                @pl.when(i > 0)
                def _(k=k, j=j):
                    pltpu.make_async_copy(
                        rows[k], acc_sh.at[dst_v.at[j]], ssems[k]).wait()

                pltpu.async_copy(g_sh.at[src_v.at[j]], rows[k], gsems[k])
            for k in range(NBUF):
                j = NBUF * i + k
                pltpu.make_async_copy(
                    g_sh.at[src_v.at[j]], rows[k], gsems[k]).wait()
                pltpu.async_copy(rows[k], acc_sh.at[dst_v.at[j]], ssems[k],
                                 add=True)
            return carry

        lax.fori_loop(0, CH_PER_TILE // NBUF, body, 0)
        for k in range(NBUF):
            pltpu.make_async_copy(rows[k], acc_sh.at[dst_v.at[0]],
                                  ssems[k]).wait()
        plsc.subcore_barrier()
        pltpu.sync_copy(
            acc_sh.at[pl.ds(s * ROWS_PER_TILE, ROWS_PER_TILE)],
            out_hbm.at[c, pl.ds(s * ROWS_PER_TILE, ROWS_PER_TILE)],
        )

    return agg_kernel(g, src_r, dst_r)


def _dot1(a, b):
    return jnp.dot(a.astype(jnp.bfloat16), b.astype(jnp.bfloat16),
                   preferred_element_type=jnp.float32)


def _dinv_from(deg_ref):
    deg = deg_ref[0, :, 0] + deg_ref[1, :, 0]
    return 1.0 / jnp.sqrt(deg + 1.0)


def _first_body(xp_ref, degp_ref, w1p_ref, g1_ref, dp_ref):
    d2 = degp_ref[0] + degp_ref[1]
    di16 = 1.0 / jnp.sqrt(d2 + 1.0)
    dinvp = jnp.concatenate(
        [jnp.broadcast_to(di16[:, 0:1], (BRP, HID)),
         jnp.broadcast_to(di16[:, DEGW:DEGW + 1], (BRP, HID))], axis=1)
    h = _dot1(xp_ref[...], w1p_ref[...])
    g1_ref[...] = h * dinvp
    dp_ref[...] = dinvp


def _tc_first(xp, degp, W1p):
    return pl.pallas_call(
        _first_body,
        grid=(NH // BRP,),
        in_specs=[
            pl.BlockSpec((BRP, 256), lambda i: (i, 0)),
            pl.BlockSpec((NC, BRP, 2 * DEGW), lambda i: (0, i, 0)),
            pl.BlockSpec((256, 128), lambda i: (0, 0)),
        ],
        out_specs=[
            pl.BlockSpec((BRP, 128), lambda i: (i, 0)),
            pl.BlockSpec((BRP, 128), lambda i: (i, 0)),
        ],
        out_shape=[
            jax.ShapeDtypeStruct((NH, 128), jnp.float32),
            jax.ShapeDtypeStruct((NH, 128), jnp.float32),
        ],
    )(xp, degp, W1p)


NH = NPAD // 2
BRP = BR // 2


def _mid_body(p_ref, g1_ref, dp_ref, w22_ref, b1_ref, out_ref):
    dp = dp_ref[...]
    pp = p_ref[0] + p_ref[1] + g1_ref[...]
    t = jnp.maximum(pp * dp + b1_ref[...], 0.0)
    g2 = _dot1(t, w22_ref[...]) * dp
    row = pl.program_id(0) * BRP + lax.broadcasted_iota(jnp.int32, (BRP, 1), 0)
    out_ref[...] = jnp.where(row < NNODES // 2, g2, 0.0)


def _tc_mid(p128, g1p, dinvp, W22, b1p):
    return pl.pallas_call(
        _mid_body,
        grid=(NH // BRP,),
        in_specs=[
            pl.BlockSpec((NC, BRP, 128), lambda i: (0, i, 0)),
            pl.BlockSpec((BRP, 128), lambda i: (i, 0)),
            pl.BlockSpec((BRP, 128), lambda i: (i, 0)),
            pl.BlockSpec((128, 128), lambda i: (0, 0)),
            pl.BlockSpec((1, 128), lambda i: (0, 0)),
        ],
        out_specs=pl.BlockSpec((BRP, 128), lambda i: (i, 0)),
        out_shape=jax.ShapeDtypeStruct((NH, 128), jnp.float32),
    )(p128, g1p, dinvp, W22, b1p)


def _final_body(q_ref, g2_ref, dp_ref, b2_ref, w3_ref, b3_ref, out_ref):
    qq = q_ref[0] + q_ref[1] + g2_ref[...]
    t = jnp.maximum(qq * dp_ref[...] + b2_ref[...], 0.0)
    tb = t.astype(jnp.bfloat16).astype(jnp.float32)
    wb = w3_ref[...].astype(jnp.bfloat16).astype(jnp.float32)
    m = tb * wb
    o_lo = jnp.sum(m[:, :HID], axis=1, keepdims=True)
    o_hi = jnp.sum(m[:, HID:], axis=1, keepdims=True)
    out_ref[...] = jnp.concatenate([o_lo, o_hi], axis=1) + b3_ref[0, 0]


def _tc_final(q128, g2p, dinvp, b2p, w3p, b3r):
    return pl.pallas_call(
        _final_body,
        grid=(NH // BRP,),
        in_specs=[
            pl.BlockSpec((NC, BRP, 128), lambda i: (0, i, 0)),
            pl.BlockSpec((BRP, 128), lambda i: (i, 0)),
            pl.BlockSpec((BRP, 128), lambda i: (i, 0)),
            pl.BlockSpec((1, 128), lambda i: (0, 0)),
            pl.BlockSpec((1, 128), lambda i: (0, 0)),
            pl.BlockSpec((1, 128), lambda i: (0, 0)),
        ],
        out_specs=pl.BlockSpec((BRP, 2), lambda i: (i, 0)),
        out_shape=jax.ShapeDtypeStruct((NH, 2), jnp.float32),
    )(q128, g2p, dinvp, b2p, w3p, b3r)


def kernel(x, edge_index, W1, b1, W2, b2, W3, b3):
    n, _ = x.shape
    e = edge_index.shape[1]
    ep = jnp.pad(edge_index, ((0, 0), (0, EPAD - e)), constant_values=n)
    src_r = ep[0].reshape(NC, NS, CH_PER_TILE, CHUNK)
    dst_r = ep[1].reshape(NC, NS, CH_PER_TILE, CHUNK)

    const8 = jnp.stack([jnp.ones((CHUNK, DEGW), jnp.float32),
                        jnp.zeros((CHUNK, DEGW), jnp.float32)])
    w3row = W3.reshape(1, HID)
    w3p = jnp.concatenate([w3row, w3row], axis=1)
    b1p = jnp.concatenate([b1, b1]).reshape(1, 128)
    b2p = jnp.concatenate([b2, b2]).reshape(1, 128)
    b3r = jnp.broadcast_to(b3.reshape(1, 1), (1, 128))
    W22 = jnp.zeros((128, 128), jnp.float32)
    W22 = W22.at[:HID, :HID].set(W2).at[HID:, HID:].set(W2)

    W1p = jnp.zeros((256, 128), jnp.float32)
    W1p = W1p.at[:128, :HID].set(W1).at[128:, HID:].set(W1)
    xp = jnp.pad(x, ((0, NPAD - n), (0, 0))).reshape(NH, 256)

    deg2 = _sc_degree(dst_r, const8)
    g1p, dinvp = _tc_first(xp, deg2.reshape(NC, NH, 2 * DEGW), W1p)
    p = _sc_aggregate(g1p.reshape(NPAD, HID), src_r, dst_r)
    g2p = _tc_mid(p.reshape(NC, NH, 128), g1p, dinvp, W22, b1p)
    q = _sc_aggregate(g2p.reshape(NPAD, HID), src_r, dst_r)
    res = _tc_final(q.reshape(NC, NH, 128), g2p, dinvp, b2p, w3p, b3r)
    return res.reshape(-1)[:n]
